# baseline (speedup 1.0000x reference)
import functools

import jax
import jax.numpy as jnp
from jax import lax
from jax.experimental import pallas as pl
from jax.experimental.pallas import tpu as pltpu
from jax.experimental.pallas import tpu_sc as plsc

_B, _D, _T, _K = 4, 64, 1024, 8192
_KC = 2048
_NKC = _K // _KC
_N = _B * _T
_MUSIC_WEIGHT = 0.1
_NW = 32
_BW = _N // _NW


def _argmin_kernel(xt_ref, sx_ref, cbt_ref, sc_ref, mc_ref, wp_ref, bp_ref,
                   codes_ref, music_ref):
    b = pl.program_id(0)

    @pl.when(b == 0)
    def _init():
        music_ref[0, 0] = 0.0

    xb = xt_ref[0]
    sx = sx_ref[0]

    xm2 = xb * (-2.0)
    iota_loc = lax.broadcasted_iota(jnp.int32, (_T, _KC), 1)
    run_min = jnp.full((_T, 1), jnp.inf, dtype=jnp.float32)
    run_idx = jnp.zeros((_T, 1), dtype=jnp.int32)
    for c in range(_NKC):
        cbt_chunk = cbt_ref[:, pl.ds(c * _KC, _KC)]
        sc = sc_ref[:, pl.ds(c * _KC, _KC)]
        mm2 = lax.dot_general(xm2, cbt_chunk, (((1,), (0,)), ((), ())),
                              preferred_element_type=jnp.float32)
        dist = jnp.sqrt(jnp.maximum((sx + sc) + mm2, 0.0))
        cmin = jnp.min(dist, axis=1, keepdims=True)
        cidx = jnp.min(jnp.where(dist == cmin, iota_loc, _K),
                       axis=1, keepdims=True) + (c * _KC)
        upd = cmin < run_min
        run_min = jnp.where(upd, cmin, run_min)
        run_idx = jnp.where(upd, cidx, run_idx)

    codes_ref[0] = run_idx

    proj = lax.dot_general(xb, wp_ref[...], (((1,), (1,)), ((), ())),
                           preferred_element_type=jnp.float32) + bp_ref[...]
    ctx = mc_ref[0]
    num = jnp.sum(proj * ctx, axis=1, keepdims=True)
    n1 = jnp.sqrt(jnp.sum(proj * proj, axis=1, keepdims=True))
    n2 = jnp.sqrt(jnp.sum(ctx * ctx, axis=1, keepdims=True))
    sim = num / (jnp.maximum(n1, 1e-8) * jnp.maximum(n2, 1e-8))
    sd = jnp.abs(sim[:-1] - sim[1:])
    cd = (run_idx[:-1] != run_idx[1:]).astype(jnp.float32)
    music_ref[0, 0] += jnp.sum(sd * cd)


def _sc_gather_kernel(codes_hbm, x_hbm, cb_hbm, zeros_hbm,
                      quant_hbm, counts_hbm, commit_hbm,
                      idx_v, rows_v, x_v, ones_v, acc_v, shared_counts, sem):
    cid = lax.axis_index("c")
    sid = lax.axis_index("s")
    wid = sid * 2 + cid
    base = wid * _BW

    pltpu.sync_copy(codes_hbm.at[pl.ds(base, _BW)], idx_v)
    for j in range(_BW // 16):
        sl = pl.ds(j * 16, 16)
        idx_v[sl] = jnp.minimum(jnp.maximum(idx_v[sl], 0), _K - 1)
    pltpu.async_copy(cb_hbm.at[idx_v], rows_v, sem).wait()
    pltpu.sync_copy(x_hbm.at[pl.ds(base, _BW)], x_v)

    for j in range(_BW // 16):
        ones_v[pl.ds(j * 16, 16)] = jnp.ones((16,), jnp.float32)

    @pl.when(sid == 0)
    def _zero():
        pltpu.sync_copy(zeros_hbm, shared_counts)

    plsc.subcore_barrier()
    pltpu.sync_copy(ones_v, shared_counts.at[idx_v], add=True)
    plsc.subcore_barrier()

    @pl.when(sid == 0)
    def _flush():
        pltpu.sync_copy(shared_counts, counts_hbm.at[cid])

    def body(i, acc):
        for j in range(_D // 16):
            sl = pl.ds(j * 16, 16)
            q = rows_v[i, sl]
            xv = x_v[i, sl]
            diff = q - xv
            x_v[i, sl] = xv + diff
            acc = acc + diff * diff
        return acc

    acc = lax.fori_loop(0, _BW, body, jnp.zeros((16,), jnp.float32))
    acc_v[...] = acc
    pltpu.sync_copy(x_v, quant_hbm.at[pl.ds(base, _BW)])
    pltpu.sync_copy(acc_v, commit_hbm.at[pl.ds(wid * 16, 16)])


def _epilogue_kernel(counts2_ref, commit_ref, music_ref,
                     ctot_ref, cbl_ref, perp_ref, mloss_ref):
    commitment = jnp.sum(commit_ref[...]) / float(_B * _D * _T)
    music_loss = music_ref[0, 0] / float(_B * (_T - 1))
    cnt = counts2_ref[0:1, :] + counts2_ref[1:2, :]
    p = cnt * (1.0 / float(_N))
    ent = jnp.sum(p * jnp.log(p + 1e-10))
    perp_ref[0, 0] = jnp.exp(-ent)
    mloss_ref[0, 0] = music_loss
    ctot_ref[0, 0] = commitment + _MUSIC_WEIGHT * music_loss
    cbl_ref[0, 0] = commitment


def _argmin_stage(x, music_context, codebook, W_proj, b_proj):
    b, d, t = x.shape
    k = codebook.shape[0]
    xt = jnp.transpose(x, (0, 2, 1))
    x_flat = xt.reshape(-1, d)
    sx = jnp.sum(x_flat ** 2, axis=1, keepdims=True).reshape(b, t, 1)
    sc = jnp.sum(codebook ** 2, axis=1).reshape(1, k)
    cbt = jnp.transpose(codebook)
    mc = jnp.transpose(music_context, (0, 2, 1))
    bp = b_proj.reshape(1, 3)

    codes3, music_sum = pl.pallas_call(
        _argmin_kernel,
        grid=(b,),
        in_specs=[
            pl.BlockSpec((1, t, d), lambda i: (i, 0, 0)),
            pl.BlockSpec((1, t, 1), lambda i: (i, 0, 0)),
            pl.BlockSpec((d, k), lambda i: (0, 0)),
            pl.BlockSpec((1, k), lambda i: (0, 0)),
            pl.BlockSpec((1, t, 3), lambda i: (i, 0, 0)),
            pl.BlockSpec((3, d), lambda i: (0, 0)),
            pl.BlockSpec((1, 3), lambda i: (0, 0)),
        ],
        out_specs=[
            pl.BlockSpec((1, t, 1), lambda i: (i, 0, 0)),
            pl.BlockSpec(memory_space=pltpu.SMEM),
        ],
        out_shape=[
            jax.ShapeDtypeStruct((b, t, 1), jnp.int32),
            jax.ShapeDtypeStruct((1, 1), jnp.float32),
        ],
    )(xt, sx, cbt, sc, mc, W_proj, bp)
    return codes3, music_sum, x_flat


def kernel(x, music_context, codebook, W_proj, b_proj):
    b, d, t = x.shape
    k = codebook.shape[0]
    codes3, music_sum, x_flat = _argmin_stage(x, music_context, codebook,
                                              W_proj, b_proj)
    codes_flat = codes3.reshape(-1)
    cb_pad = jnp.pad(codebook, ((0, 0), (0, 128 - d)))
    x_pad = jnp.pad(x_flat, ((0, 0), (0, 128 - d)))

    sc_call = functools.partial(
        pl.kernel,
        mesh=plsc.VectorSubcoreMesh(core_axis_name="c", subcore_axis_name="s"),
        out_type=[
            jax.ShapeDtypeStruct((_N, 128), jnp.float32),
            jax.ShapeDtypeStruct((2, k), jnp.float32),
            jax.ShapeDtypeStruct((_NW * 16,), jnp.float32),
        ],
        scratch_types=[
            pltpu.VMEM((_BW,), jnp.int32),
            pltpu.VMEM((_BW, 128), jnp.float32),
            pltpu.VMEM((_BW, 128), jnp.float32),
            pltpu.VMEM((_BW,), jnp.float32),
            pltpu.VMEM((16,), jnp.float32),
            pltpu.VMEM_SHARED((k,), jnp.float32),
            pltpu.SemaphoreType.DMA,
        ],
    )(_sc_gather_kernel)
    quant_pad, counts2, commit_parts = sc_call(
        codes_flat, x_pad, cb_pad, jnp.zeros((k,), jnp.float32))
    quant_flat = quant_pad[:, :d]

    commit_parts = commit_parts.reshape(_NW, 16)
    ctot, cbl, perp, mloss = pl.pallas_call(
        _epilogue_kernel,
        in_specs=[
            pl.BlockSpec(memory_space=pltpu.VMEM),
            pl.BlockSpec(memory_space=pltpu.VMEM),
            pl.BlockSpec(memory_space=pltpu.SMEM),
        ],
        out_specs=[
            pl.BlockSpec(memory_space=pltpu.SMEM),
            pl.BlockSpec(memory_space=pltpu.SMEM),
            pl.BlockSpec(memory_space=pltpu.SMEM),
            pl.BlockSpec(memory_space=pltpu.SMEM),
        ],
        out_shape=[
            jax.ShapeDtypeStruct((1, 1), jnp.float32),
            jax.ShapeDtypeStruct((1, 1), jnp.float32),
            jax.ShapeDtypeStruct((1, 1), jnp.float32),
            jax.ShapeDtypeStruct((1, 1), jnp.float32),
        ],
    )(counts2, commit_parts, music_sum)

    quantized_st = jnp.transpose(quant_flat.reshape(b, t, d), (0, 2, 1))
    codes_bt = codes3.reshape(b, t)
    return (quantized_st, codes_bt, ctot.reshape(()), cbl.reshape(()),
            perp.reshape(()), mloss.reshape(()))


# speedup vs baseline: 1.0135x; 1.0135x over previous
import functools

import jax
import jax.numpy as jnp
from jax import lax
from jax.experimental import pallas as pl
from jax.experimental.pallas import tpu as pltpu
from jax.experimental.pallas import tpu_sc as plsc

_B, _D, _T, _K = 4, 64, 1024, 8192
_KC = 4096
_NKC = _K // _KC
_N = _B * _T
_MUSIC_WEIGHT = 0.1
_NW = 32
_BW = _N // _NW


def _argmin_kernel(xt_ref, sx_ref, cbt_ref, sc_ref, mc_ref, wp_ref, bp_ref,
                   codes_ref, music_ref):
    b = pl.program_id(0)

    @pl.when(b == 0)
    def _init():
        music_ref[0, 0] = 0.0

    xb = xt_ref[0]
    sx = sx_ref[0]

    xm2 = xb * (-2.0)
    iota_loc = lax.broadcasted_iota(jnp.int32, (_T, _KC), 1)
    run_min = jnp.full((_T, 1), jnp.inf, dtype=jnp.float32)
    run_idx = jnp.zeros((_T, 1), dtype=jnp.int32)
    for c in range(_NKC):
        cbt_chunk = cbt_ref[:, pl.ds(c * _KC, _KC)]
        sc = sc_ref[:, pl.ds(c * _KC, _KC)]
        mm2 = lax.dot_general(xm2, cbt_chunk, (((1,), (0,)), ((), ())),
                              preferred_element_type=jnp.float32)
        dist = jnp.sqrt(jnp.maximum((sx + sc) + mm2, 0.0))
        cmin = jnp.min(dist, axis=1, keepdims=True)
        cidx = jnp.min(jnp.where(dist == cmin, iota_loc, _K),
                       axis=1, keepdims=True) + (c * _KC)
        upd = cmin < run_min
        run_min = jnp.where(upd, cmin, run_min)
        run_idx = jnp.where(upd, cidx, run_idx)

    codes_ref[0] = run_idx

    proj = lax.dot_general(xb, wp_ref[...], (((1,), (1,)), ((), ())),
                           preferred_element_type=jnp.float32) + bp_ref[...]
    ctx = mc_ref[0]
    num = jnp.sum(proj * ctx, axis=1, keepdims=True)
    n1 = jnp.sqrt(jnp.sum(proj * proj, axis=1, keepdims=True))
    n2 = jnp.sqrt(jnp.sum(ctx * ctx, axis=1, keepdims=True))
    sim = num / (jnp.maximum(n1, 1e-8) * jnp.maximum(n2, 1e-8))
    sd = jnp.abs(sim[:-1] - sim[1:])
    cd = (run_idx[:-1] != run_idx[1:]).astype(jnp.float32)
    music_ref[0, 0] += jnp.sum(sd * cd)


def _sc_gather_kernel(codes_hbm, x_hbm, cb_hbm, zeros_hbm,
                      quant_hbm, counts_hbm, commit_hbm,
                      idx_v, rows_v, x_v, ones_v, acc_v, shared_counts, sem):
    cid = lax.axis_index("c")
    sid = lax.axis_index("s")
    wid = sid * 2 + cid
    base = wid * _BW

    pltpu.sync_copy(codes_hbm.at[pl.ds(base, _BW)], idx_v)
    for j in range(_BW // 16):
        sl = pl.ds(j * 16, 16)
        idx_v[sl] = jnp.minimum(jnp.maximum(idx_v[sl], 0), _K - 1)
    pltpu.async_copy(cb_hbm.at[idx_v], rows_v, sem).wait()
    pltpu.sync_copy(x_hbm.at[pl.ds(base, _BW)], x_v)

    for j in range(_BW // 16):
        ones_v[pl.ds(j * 16, 16)] = jnp.ones((16,), jnp.float32)

    @pl.when(sid == 0)
    def _zero():
        pltpu.sync_copy(zeros_hbm, shared_counts)

    plsc.subcore_barrier()
    pltpu.sync_copy(ones_v, shared_counts.at[idx_v], add=True)
    plsc.subcore_barrier()

    @pl.when(sid == 0)
    def _flush():
        pltpu.sync_copy(shared_counts, counts_hbm.at[cid])

    def body(i, acc):
        for j in range(_D // 16):
            sl = pl.ds(j * 16, 16)
            q = rows_v[i, sl]
            xv = x_v[i, sl]
            diff = q - xv
            x_v[i, sl] = xv + diff
            acc = acc + diff * diff
        return acc

    acc = lax.fori_loop(0, _BW, body, jnp.zeros((16,), jnp.float32))
    acc_v[...] = acc
    pltpu.sync_copy(x_v, quant_hbm.at[pl.ds(base, _BW)])
    pltpu.sync_copy(acc_v, commit_hbm.at[pl.ds(wid * 16, 16)])


def _epilogue_kernel(counts2_ref, commit_ref, music_ref,
                     ctot_ref, cbl_ref, perp_ref, mloss_ref):
    commitment = jnp.sum(commit_ref[...]) / float(_B * _D * _T)
    music_loss = music_ref[0, 0] / float(_B * (_T - 1))
    cnt = counts2_ref[0:1, :] + counts2_ref[1:2, :]
    p = cnt * (1.0 / float(_N))
    ent = jnp.sum(p * jnp.log(p + 1e-10))
    perp_ref[0, 0] = jnp.exp(-ent)
    mloss_ref[0, 0] = music_loss
    ctot_ref[0, 0] = commitment + _MUSIC_WEIGHT * music_loss
    cbl_ref[0, 0] = commitment


def _argmin_stage(x, music_context, codebook, W_proj, b_proj):
    b, d, t = x.shape
    k = codebook.shape[0]
    xt = jnp.transpose(x, (0, 2, 1))
    x_flat = xt.reshape(-1, d)
    sx = jnp.sum(x_flat ** 2, axis=1, keepdims=True).reshape(b, t, 1)
    sc = jnp.sum(codebook ** 2, axis=1).reshape(1, k)
    cbt = jnp.transpose(codebook)
    mc = jnp.transpose(music_context, (0, 2, 1))
    bp = b_proj.reshape(1, 3)

    codes3, music_sum = pl.pallas_call(
        _argmin_kernel,
        grid=(b,),
        in_specs=[
            pl.BlockSpec((1, t, d), lambda i: (i, 0, 0)),
            pl.BlockSpec((1, t, 1), lambda i: (i, 0, 0)),
            pl.BlockSpec((d, k), lambda i: (0, 0)),
            pl.BlockSpec((1, k), lambda i: (0, 0)),
            pl.BlockSpec((1, t, 3), lambda i: (i, 0, 0)),
            pl.BlockSpec((3, d), lambda i: (0, 0)),
            pl.BlockSpec((1, 3), lambda i: (0, 0)),
        ],
        out_specs=[
            pl.BlockSpec((1, t, 1), lambda i: (i, 0, 0)),
            pl.BlockSpec(memory_space=pltpu.SMEM),
        ],
        out_shape=[
            jax.ShapeDtypeStruct((b, t, 1), jnp.int32),
            jax.ShapeDtypeStruct((1, 1), jnp.float32),
        ],
    )(xt, sx, cbt, sc, mc, W_proj, bp)
    return codes3, music_sum, x_flat


def kernel(x, music_context, codebook, W_proj, b_proj):
    b, d, t = x.shape
    k = codebook.shape[0]
    codes3, music_sum, x_flat = _argmin_stage(x, music_context, codebook,
                                              W_proj, b_proj)
    codes_flat = codes3.reshape(-1)
    cb_pad = jnp.pad(codebook, ((0, 0), (0, 128 - d)))
    x_pad = jnp.pad(x_flat, ((0, 0), (0, 128 - d)))

    sc_call = functools.partial(
        pl.kernel,
        mesh=plsc.VectorSubcoreMesh(core_axis_name="c", subcore_axis_name="s"),
        out_type=[
            jax.ShapeDtypeStruct((_N, 128), jnp.float32),
            jax.ShapeDtypeStruct((2, k), jnp.float32),
            jax.ShapeDtypeStruct((_NW * 16,), jnp.float32),
        ],
        scratch_types=[
            pltpu.VMEM((_BW,), jnp.int32),
            pltpu.VMEM((_BW, 128), jnp.float32),
            pltpu.VMEM((_BW, 128), jnp.float32),
            pltpu.VMEM((_BW,), jnp.float32),
            pltpu.VMEM((16,), jnp.float32),
            pltpu.VMEM_SHARED((k,), jnp.float32),
            pltpu.SemaphoreType.DMA,
        ],
    )(_sc_gather_kernel)
    quant_pad, counts2, commit_parts = sc_call(
        codes_flat, x_pad, cb_pad, jnp.zeros((k,), jnp.float32))
    quant_flat = quant_pad[:, :d]

    commit_parts = commit_parts.reshape(_NW, 16)
    ctot, cbl, perp, mloss = pl.pallas_call(
        _epilogue_kernel,
        in_specs=[
            pl.BlockSpec(memory_space=pltpu.VMEM),
            pl.BlockSpec(memory_space=pltpu.VMEM),
            pl.BlockSpec(memory_space=pltpu.SMEM),
        ],
        out_specs=[
            pl.BlockSpec(memory_space=pltpu.SMEM),
            pl.BlockSpec(memory_space=pltpu.SMEM),
            pl.BlockSpec(memory_space=pltpu.SMEM),
            pl.BlockSpec(memory_space=pltpu.SMEM),
        ],
        out_shape=[
            jax.ShapeDtypeStruct((1, 1), jnp.float32),
            jax.ShapeDtypeStruct((1, 1), jnp.float32),
            jax.ShapeDtypeStruct((1, 1), jnp.float32),
            jax.ShapeDtypeStruct((1, 1), jnp.float32),
        ],
    )(counts2, commit_parts, music_sum)

    quantized_st = jnp.transpose(quant_flat.reshape(b, t, d), (0, 2, 1))
    codes_bt = codes3.reshape(b, t)
    return (quantized_st, codes_bt, ctot.reshape(()), cbl.reshape(()),
            perp.reshape(()), mloss.reshape(()))


# speedup vs baseline: 1.0220x; 1.0084x over previous
import functools

import jax
import jax.numpy as jnp
from jax import lax
from jax.experimental import pallas as pl
from jax.experimental.pallas import tpu as pltpu
from jax.experimental.pallas import tpu_sc as plsc

_B, _D, _T, _K = 4, 64, 1024, 8192
_KC = 8192
_NKC = _K // _KC
_N = _B * _T
_MUSIC_WEIGHT = 0.1
_NW = 32
_BW = _N // _NW


def _argmin_kernel(xt_ref, sx_ref, cbt_ref, sc_ref, mc_ref, wp_ref, bp_ref,
                   codes_ref, music_ref):
    b = pl.program_id(0)

    @pl.when(b == 0)
    def _init():
        music_ref[0, 0] = 0.0

    xb = xt_ref[0]
    sx = sx_ref[0]

    xm2 = xb * (-2.0)
    iota_loc = lax.broadcasted_iota(jnp.int32, (_T, _KC), 1)
    run_min = jnp.full((_T, 1), jnp.inf, dtype=jnp.float32)
    run_idx = jnp.zeros((_T, 1), dtype=jnp.int32)
    for c in range(_NKC):
        cbt_chunk = cbt_ref[:, pl.ds(c * _KC, _KC)]
        sc = sc_ref[:, pl.ds(c * _KC, _KC)]
        mm2 = lax.dot_general(xm2, cbt_chunk, (((1,), (0,)), ((), ())),
                              preferred_element_type=jnp.float32)
        dist = jnp.sqrt(jnp.maximum((sx + sc) + mm2, 0.0))
        cmin = jnp.min(dist, axis=1, keepdims=True)
        cidx = jnp.min(jnp.where(dist == cmin, iota_loc, _K),
                       axis=1, keepdims=True) + (c * _KC)
        upd = cmin < run_min
        run_min = jnp.where(upd, cmin, run_min)
        run_idx = jnp.where(upd, cidx, run_idx)

    codes_ref[0] = run_idx

    proj = lax.dot_general(xb, wp_ref[...], (((1,), (1,)), ((), ())),
                           preferred_element_type=jnp.float32) + bp_ref[...]
    ctx = mc_ref[0]
    num = jnp.sum(proj * ctx, axis=1, keepdims=True)
    n1 = jnp.sqrt(jnp.sum(proj * proj, axis=1, keepdims=True))
    n2 = jnp.sqrt(jnp.sum(ctx * ctx, axis=1, keepdims=True))
    sim = num / (jnp.maximum(n1, 1e-8) * jnp.maximum(n2, 1e-8))
    sd = jnp.abs(sim[:-1] - sim[1:])
    cd = (run_idx[:-1] != run_idx[1:]).astype(jnp.float32)
    music_ref[0, 0] += jnp.sum(sd * cd)


def _sc_gather_kernel(codes_hbm, x_hbm, cb_hbm, zeros_hbm,
                      quant_hbm, counts_hbm, commit_hbm,
                      idx_v, rows_v, x_v, ones_v, acc_v, shared_counts, sem):
    cid = lax.axis_index("c")
    sid = lax.axis_index("s")
    wid = sid * 2 + cid
    base = wid * _BW

    pltpu.sync_copy(codes_hbm.at[pl.ds(base, _BW)], idx_v)
    for j in range(_BW // 16):
        sl = pl.ds(j * 16, 16)
        idx_v[sl] = jnp.minimum(jnp.maximum(idx_v[sl], 0), _K - 1)
    pltpu.async_copy(cb_hbm.at[idx_v], rows_v, sem).wait()
    pltpu.sync_copy(x_hbm.at[pl.ds(base, _BW)], x_v)

    for j in range(_BW // 16):
        ones_v[pl.ds(j * 16, 16)] = jnp.ones((16,), jnp.float32)

    @pl.when(sid == 0)
    def _zero():
        pltpu.sync_copy(zeros_hbm, shared_counts)

    plsc.subcore_barrier()
    pltpu.sync_copy(ones_v, shared_counts.at[idx_v], add=True)
    plsc.subcore_barrier()

    @pl.when(sid == 0)
    def _flush():
        pltpu.sync_copy(shared_counts, counts_hbm.at[cid])

    def body(i, acc):
        for j in range(_D // 16):
            sl = pl.ds(j * 16, 16)
            q = rows_v[i, sl]
            xv = x_v[i, sl]
            diff = q - xv
            x_v[i, sl] = xv + diff
            acc = acc + diff * diff
        return acc

    acc = lax.fori_loop(0, _BW, body, jnp.zeros((16,), jnp.float32))
    acc_v[...] = acc
    pltpu.sync_copy(x_v, quant_hbm.at[pl.ds(base, _BW)])
    pltpu.sync_copy(acc_v, commit_hbm.at[pl.ds(wid * 16, 16)])


def _epilogue_kernel(counts2_ref, commit_ref, music_ref,
                     ctot_ref, cbl_ref, perp_ref, mloss_ref):
    commitment = jnp.sum(commit_ref[...]) / float(_B * _D * _T)
    music_loss = music_ref[0, 0] / float(_B * (_T - 1))
    cnt = counts2_ref[0:1, :] + counts2_ref[1:2, :]
    p = cnt * (1.0 / float(_N))
    ent = jnp.sum(p * jnp.log(p + 1e-10))
    perp_ref[0, 0] = jnp.exp(-ent)
    mloss_ref[0, 0] = music_loss
    ctot_ref[0, 0] = commitment + _MUSIC_WEIGHT * music_loss
    cbl_ref[0, 0] = commitment


def _argmin_stage(x, music_context, codebook, W_proj, b_proj):
    b, d, t = x.shape
    k = codebook.shape[0]
    xt = jnp.transpose(x, (0, 2, 1))
    x_flat = xt.reshape(-1, d)
    sx = jnp.sum(x_flat ** 2, axis=1, keepdims=True).reshape(b, t, 1)
    sc = jnp.sum(codebook ** 2, axis=1).reshape(1, k)
    cbt = jnp.transpose(codebook)
    mc = jnp.transpose(music_context, (0, 2, 1))
    bp = b_proj.reshape(1, 3)

    codes3, music_sum = pl.pallas_call(
        _argmin_kernel,
        grid=(b,),
        in_specs=[
            pl.BlockSpec((1, t, d), lambda i: (i, 0, 0)),
            pl.BlockSpec((1, t, 1), lambda i: (i, 0, 0)),
            pl.BlockSpec((d, k), lambda i: (0, 0)),
            pl.BlockSpec((1, k), lambda i: (0, 0)),
            pl.BlockSpec((1, t, 3), lambda i: (i, 0, 0)),
            pl.BlockSpec((3, d), lambda i: (0, 0)),
            pl.BlockSpec((1, 3), lambda i: (0, 0)),
        ],
        out_specs=[
            pl.BlockSpec((1, t, 1), lambda i: (i, 0, 0)),
            pl.BlockSpec(memory_space=pltpu.SMEM),
        ],
        out_shape=[
            jax.ShapeDtypeStruct((b, t, 1), jnp.int32),
            jax.ShapeDtypeStruct((1, 1), jnp.float32),
        ],
    )(xt, sx, cbt, sc, mc, W_proj, bp)
    return codes3, music_sum, x_flat


def kernel(x, music_context, codebook, W_proj, b_proj):
    b, d, t = x.shape
    k = codebook.shape[0]
    codes3, music_sum, x_flat = _argmin_stage(x, music_context, codebook,
                                              W_proj, b_proj)
    codes_flat = codes3.reshape(-1)
    cb_pad = jnp.pad(codebook, ((0, 0), (0, 128 - d)))
    x_pad = jnp.pad(x_flat, ((0, 0), (0, 128 - d)))

    sc_call = functools.partial(
        pl.kernel,
        mesh=plsc.VectorSubcoreMesh(core_axis_name="c", subcore_axis_name="s"),
        out_type=[
            jax.ShapeDtypeStruct((_N, 128), jnp.float32),
            jax.ShapeDtypeStruct((2, k), jnp.float32),
            jax.ShapeDtypeStruct((_NW * 16,), jnp.float32),
        ],
        scratch_types=[
            pltpu.VMEM((_BW,), jnp.int32),
            pltpu.VMEM((_BW, 128), jnp.float32),
            pltpu.VMEM((_BW, 128), jnp.float32),
            pltpu.VMEM((_BW,), jnp.float32),
            pltpu.VMEM((16,), jnp.float32),
            pltpu.VMEM_SHARED((k,), jnp.float32),
            pltpu.SemaphoreType.DMA,
        ],
    )(_sc_gather_kernel)
    quant_pad, counts2, commit_parts = sc_call(
        codes_flat, x_pad, cb_pad, jnp.zeros((k,), jnp.float32))
    quant_flat = quant_pad[:, :d]

    commit_parts = commit_parts.reshape(_NW, 16)
    ctot, cbl, perp, mloss = pl.pallas_call(
        _epilogue_kernel,
        in_specs=[
            pl.BlockSpec(memory_space=pltpu.VMEM),
            pl.BlockSpec(memory_space=pltpu.VMEM),
            pl.BlockSpec(memory_space=pltpu.SMEM),
        ],
        out_specs=[
            pl.BlockSpec(memory_space=pltpu.SMEM),
            pl.BlockSpec(memory_space=pltpu.SMEM),
            pl.BlockSpec(memory_space=pltpu.SMEM),
            pl.BlockSpec(memory_space=pltpu.SMEM),
        ],
        out_shape=[
            jax.ShapeDtypeStruct((1, 1), jnp.float32),
            jax.ShapeDtypeStruct((1, 1), jnp.float32),
            jax.ShapeDtypeStruct((1, 1), jnp.float32),
            jax.ShapeDtypeStruct((1, 1), jnp.float32),
        ],
    )(counts2, commit_parts, music_sum)

    quantized_st = jnp.transpose(quant_flat.reshape(b, t, d), (0, 2, 1))
    codes_bt = codes3.reshape(b, t)
    return (quantized_st, codes_bt, ctot.reshape(()), cbl.reshape(()),
            perp.reshape(()), mloss.reshape(()))
